# baseline (speedup 1.0000x reference)
import functools

import jax
import jax.numpy as jnp
from jax import lax
from jax.experimental import pallas as pl
from jax.experimental.pallas import tpu as pltpu
from jax.experimental.pallas import tpu_sc as plsc

_N = 10000
_E = 320000
_FCAT = 96
_NTILES = 32
_EPT = _E // _NTILES
_FPT = _FCAT // _NTILES
_CHUNK = 10000
_L = 16


def _tile_id():
    return lax.axis_index("s") * 2 + lax.axis_index("c")


def _zero_vmem(ref, nwords):
    z = jnp.zeros((_L,), ref.dtype)

    @plsc.parallel_loop(0, nwords // _L, unroll=8)
    def _(i):
        ref[pl.ds(i * _L, _L)] = z


def _deg_body(g_hbm, ew_hbm, out_hbm, col_v, ew_v, deg_v):
    wid = _tile_id()
    base = wid * _EPT
    pltpu.sync_copy(g_hbm.at[pl.ds(_E + base, _EPT)], col_v)
    pltpu.sync_copy(ew_hbm.at[pl.ds(base, _EPT)], ew_v)
    _zero_vmem(deg_v, _N)

    @plsc.parallel_loop(0, _EPT // _L, unroll=8)
    def _(i):
        c16 = col_v[pl.ds(i * _L, _L)]
        w16 = ew_v[pl.ds(i * _L, _L)]
        plsc.addupdate_scatter(deg_v, [c16], w16)
    pltpu.sync_copy(deg_v, out_hbm.at[wid])


def _accum_body(g_hbm, ew_hbm, yt_hbm, out_hbm,
                y_v, acc_v, row_v0, col_v0, ew_v0, row_v1, col_v1, ew_v1, sem):
    wid = _tile_id()
    fbase = wid * _FPT * _N
    nch = _E // _CHUNK
    bufs = ((row_v0, col_v0, ew_v0), (row_v1, col_v1, ew_v1))

    def issue(ci, b):
        cbase = ci * _CHUNK
        rv, cv, wv = bufs[b]
        pltpu.async_copy(g_hbm.at[pl.ds(cbase, _CHUNK)], rv, sem.at[b])
        pltpu.async_copy(g_hbm.at[pl.ds(_E + cbase, _CHUNK)], cv, sem.at[b])
        pltpu.async_copy(ew_hbm.at[pl.ds(cbase, _CHUNK)], wv, sem.at[b])

    def drain(ci, b):
        cbase = ci * _CHUNK
        rv, cv, wv = bufs[b]
        pltpu.make_async_copy(g_hbm.at[pl.ds(cbase, _CHUNK)], rv,
                              sem.at[b]).wait()
        pltpu.make_async_copy(g_hbm.at[pl.ds(_E + cbase, _CHUNK)], cv,
                              sem.at[b]).wait()
        pltpu.make_async_copy(ew_hbm.at[pl.ds(cbase, _CHUNK)], wv,
                              sem.at[b]).wait()

    issue(0, 0)
    pltpu.sync_copy(yt_hbm.at[pl.ds(fbase, _FPT * _N)], y_v)
    _zero_vmem(acc_v, _FPT * _N)

    def outer(cc, carry):
        for b in range(2):
            ci = cc + b
            drain(ci, b)

            @pl.when(ci + 1 < nch)
            def _():
                issue(ci + 1, (b + 1) % 2)

            rv, cv, wv = bufs[b]

            @plsc.parallel_loop(0, _CHUNK // _L, unroll=10)
            def _(i):
                r16 = rv[pl.ds(i * _L, _L)]
                c16 = cv[pl.ds(i * _L, _L)]
                w16 = wv[pl.ds(i * _L, _L)]
                for f in range(_FPT):
                    v = plsc.load_gather(y_v, [r16 + f * _N])
                    plsc.addupdate_scatter(acc_v, [c16 + f * _N], v * w16)

        return carry

    lax.fori_loop(0, nch // 2, lambda j, c: outer(j * 2, c), None)
    pltpu.sync_copy(acc_v, out_hbm.at[pl.ds(fbase, _FPT * _N)])


def _dense1_body(degp_ref, wz_ref, wr_ref, wh_ref, x_ref, yt_ref, dinv_ref):
    deg = jnp.sum(degp_ref[...], axis=0, keepdims=True)
    pos = deg > 0.0
    dinv = jnp.where(pos, lax.rsqrt(jnp.where(pos, deg, 1.0)), 0.0)
    dinv_ref[...] = dinv
    x = x_ref[...]
    for i, w_ref in enumerate((wz_ref, wr_ref, wh_ref)):
        xwT = lax.dot_general(w_ref[...], x, (((0,), (1,)), ((), ())),
                              preferred_element_type=jnp.float32)
        yt_ref[i * 32:(i + 1) * 32, :] = xwT * dinv


def _dense2_body(convT_ref, dinv_ref, hT_ref,
                 azT, bzT, arT, brT, ahT, bhT,
                 bcz, bcr, bch, blz, blr, blh,
                 woT, bout, hnT_ref, yT_ref):
    dinv = dinv_ref[...]
    hT = hT_ref[...]
    czT = convT_ref[0:32, :] * dinv + bcz[...]
    crT = convT_ref[32:64, :] * dinv + bcr[...]
    chT = convT_ref[64:96, :] * dinv + bch[...]
    zT = jax.nn.sigmoid(azT[...] @ czT + bzT[...] @ hT + blz[...])
    rT = jax.nn.sigmoid(arT[...] @ crT + brT[...] @ hT + blr[...])
    htT = jnp.tanh(ahT[...] @ chT + bhT[...] @ (hT * rT) + blh[...])
    hnT = zT * hT + (1.0 - zT) * htT
    hnT_ref[...] = hnT
    yT_ref[...] = woT[...] @ jnp.maximum(hnT, 0.0) + bout[...]


def kernel(g, node_feat, edge_weight, hidden_state,
           W_conv_z, b_conv_z, W_lin_z, b_lin_z,
           W_conv_r, b_conv_r, W_lin_r, b_lin_r,
           W_conv_h, b_conv_h, W_lin_h, b_lin_h,
           W_out, b_out):
    g32 = g.astype(jnp.int32).reshape(-1)
    ew = edge_weight.astype(jnp.float32)

    mesh = plsc.VectorSubcoreMesh(core_axis_name="c", subcore_axis_name="s")
    sc_params = pltpu.CompilerParams(needs_layout_passes=False)

    deg_parts = pl.kernel(
        _deg_body,
        out_type=jax.ShapeDtypeStruct((_NTILES, _N), jnp.float32),
        mesh=mesh,
        compiler_params=sc_params,
        scratch_types=[
            pltpu.VMEM((_EPT,), jnp.int32),
            pltpu.VMEM((_EPT,), jnp.float32),
            pltpu.VMEM((_N,), jnp.float32),
        ],
    )(g32, ew)

    yt, dinv = pl.pallas_call(
        _dense1_body,
        out_shape=(
            jax.ShapeDtypeStruct((_FCAT, _N), jnp.float32),
            jax.ShapeDtypeStruct((1, _N), jnp.float32),
        ),
    )(deg_parts, W_conv_z, W_conv_r, W_conv_h, node_feat)

    convT_flat = pl.kernel(
        _accum_body,
        out_type=jax.ShapeDtypeStruct((_FCAT * _N,), jnp.float32),
        mesh=mesh,
        compiler_params=sc_params,
        scratch_types=[
            pltpu.VMEM((_FPT * _N,), jnp.float32),
            pltpu.VMEM((_FPT * _N,), jnp.float32),
            pltpu.VMEM((_CHUNK,), jnp.int32),
            pltpu.VMEM((_CHUNK,), jnp.int32),
            pltpu.VMEM((_CHUNK,), jnp.float32),
            pltpu.VMEM((_CHUNK,), jnp.int32),
            pltpu.VMEM((_CHUNK,), jnp.int32),
            pltpu.VMEM((_CHUNK,), jnp.float32),
            pltpu.SemaphoreType.DMA((2,)),
        ],
    )(g32, ew, yt.reshape(-1))

    hnT, yT = pl.pallas_call(
        _dense2_body,
        out_shape=(
            jax.ShapeDtypeStruct((32, _N), jnp.float32),
            jax.ShapeDtypeStruct((1, _N), jnp.float32),
        ),
    )(
        convT_flat.reshape(_FCAT, _N), dinv, hidden_state.T,
        W_lin_z[:32].T, W_lin_z[32:].T,
        W_lin_r[:32].T, W_lin_r[32:].T,
        W_lin_h[:32].T, W_lin_h[32:].T,
        b_conv_z.reshape(32, 1), b_conv_r.reshape(32, 1), b_conv_h.reshape(32, 1),
        b_lin_z.reshape(32, 1), b_lin_r.reshape(32, 1), b_lin_h.reshape(32, 1),
        W_out.T, b_out.reshape(1, 1),
    )

    return (yT.T, hnT.T)


# speedup vs baseline: 1.0035x; 1.0035x over previous
import jax
import jax.numpy as jnp
from jax import lax
from jax.experimental import pallas as pl
from jax.experimental.pallas import tpu as pltpu
from jax.experimental.pallas import tpu_sc as plsc

_N = 10000
_E = 320000
_FCAT = 96
_NTILES = 32
_EPT = _E // _NTILES
_FPT = _FCAT // _NTILES
_CHUNK = 10000
_L = 16


def _tile_id():
    return lax.axis_index("s") * 2 + lax.axis_index("c")


def _zero_vmem(ref, nwords):
    z = jnp.zeros((_L,), ref.dtype)

    @plsc.parallel_loop(0, nwords // _L, unroll=8)
    def _(i):
        ref[pl.ds(i * _L, _L)] = z


def _deg_body(g_hbm, ew_hbm, out_hbm, col_v, ew_v, deg_v):
    wid = _tile_id()
    base = wid * _EPT
    pltpu.sync_copy(g_hbm.at[pl.ds(_E + base, _EPT)], col_v)
    pltpu.sync_copy(ew_hbm.at[pl.ds(base, _EPT)], ew_v)
    _zero_vmem(deg_v, _N)

    @plsc.parallel_loop(0, _EPT // _L, unroll=8)
    def _(i):
        c16 = col_v[pl.ds(i * _L, _L)]
        w16 = ew_v[pl.ds(i * _L, _L)]
        plsc.addupdate_scatter(deg_v, [c16], w16)
    pltpu.sync_copy(deg_v, out_hbm.at[wid])


def _accum_body(g_hbm, ew_hbm, yt_hbm, out_hbm,
                y_v, acc_v, row_v0, col_v0, ew_v0, row_v1, col_v1, ew_v1, sem):
    wid = _tile_id()
    fbase = wid * _FPT * _N
    nch = _E // _CHUNK
    bufs = ((row_v0, col_v0, ew_v0), (row_v1, col_v1, ew_v1))

    def issue(ci, b):
        cbase = ci * _CHUNK
        rv, cv, wv = bufs[b]
        pltpu.async_copy(g_hbm.at[pl.ds(cbase, _CHUNK)], rv, sem.at[b])
        pltpu.async_copy(g_hbm.at[pl.ds(_E + cbase, _CHUNK)], cv, sem.at[b])
        pltpu.async_copy(ew_hbm.at[pl.ds(cbase, _CHUNK)], wv, sem.at[b])

    def drain(ci, b):
        cbase = ci * _CHUNK
        rv, cv, wv = bufs[b]
        pltpu.make_async_copy(g_hbm.at[pl.ds(cbase, _CHUNK)], rv,
                              sem.at[b]).wait()
        pltpu.make_async_copy(g_hbm.at[pl.ds(_E + cbase, _CHUNK)], cv,
                              sem.at[b]).wait()
        pltpu.make_async_copy(ew_hbm.at[pl.ds(cbase, _CHUNK)], wv,
                              sem.at[b]).wait()

    issue(0, 0)
    pltpu.sync_copy(yt_hbm.at[pl.ds(fbase, _FPT * _N)], y_v)
    _zero_vmem(acc_v, _FPT * _N)

    def outer(cc, carry):
        for b in range(2):
            ci = cc + b
            drain(ci, b)

            @pl.when(ci + 1 < nch)
            def _():
                issue(ci + 1, (b + 1) % 2)

            rv, cv, wv = bufs[b]

            @plsc.parallel_loop(0, _CHUNK // _L, unroll=8)
            def _(i):
                r16 = rv[pl.ds(i * _L, _L)]
                c16 = cv[pl.ds(i * _L, _L)]
                w16 = wv[pl.ds(i * _L, _L)]
                for f in range(_FPT):
                    v = plsc.load_gather(y_v, [r16 + f * _N])
                    plsc.addupdate_scatter(acc_v, [c16 + f * _N], v * w16)

        return carry

    lax.fori_loop(0, nch // 2, lambda j, c: outer(j * 2, c), None)
    pltpu.sync_copy(acc_v, out_hbm.at[pl.ds(fbase, _FPT * _N)])


def _dense1_body(degp_ref, wz_ref, wr_ref, wh_ref, x_ref, yt_ref, dinv_ref):
    deg = jnp.sum(degp_ref[...], axis=0, keepdims=True)
    pos = deg > 0.0
    dinv = jnp.where(pos, lax.rsqrt(jnp.where(pos, deg, 1.0)), 0.0)
    dinv_ref[...] = dinv
    x = x_ref[...]
    for i, w_ref in enumerate((wz_ref, wr_ref, wh_ref)):
        xwT = lax.dot_general(w_ref[...], x, (((0,), (1,)), ((), ())),
                              preferred_element_type=jnp.float32)
        yt_ref[i * 32:(i + 1) * 32, :] = xwT * dinv


def _dense2_body(convT_ref, dinv_ref, hT_ref,
                 azT, bzT, arT, brT, ahT, bhT,
                 bcz, bcr, bch, blz, blr, blh,
                 woT, bout, hnT_ref, yT_ref):
    dinv = dinv_ref[...]
    hT = hT_ref[...]
    czT = convT_ref[0:32, :] * dinv + bcz[...]
    crT = convT_ref[32:64, :] * dinv + bcr[...]
    chT = convT_ref[64:96, :] * dinv + bch[...]
    zT = jax.nn.sigmoid(azT[...] @ czT + bzT[...] @ hT + blz[...])
    rT = jax.nn.sigmoid(arT[...] @ crT + brT[...] @ hT + blr[...])
    htT = jnp.tanh(ahT[...] @ chT + bhT[...] @ (hT * rT) + blh[...])
    hnT = zT * hT + (1.0 - zT) * htT
    hnT_ref[...] = hnT
    yT_ref[...] = woT[...] @ jnp.maximum(hnT, 0.0) + bout[...]


def kernel(g, node_feat, edge_weight, hidden_state,
           W_conv_z, b_conv_z, W_lin_z, b_lin_z,
           W_conv_r, b_conv_r, W_lin_r, b_lin_r,
           W_conv_h, b_conv_h, W_lin_h, b_lin_h,
           W_out, b_out):
    g32 = g.astype(jnp.int32).reshape(-1)
    ew = edge_weight.astype(jnp.float32)

    mesh = plsc.VectorSubcoreMesh(core_axis_name="c", subcore_axis_name="s")
    sc_params = pltpu.CompilerParams(needs_layout_passes=False)

    deg_parts = pl.kernel(
        _deg_body,
        out_type=jax.ShapeDtypeStruct((_NTILES, _N), jnp.float32),
        mesh=mesh,
        compiler_params=sc_params,
        scratch_types=[
            pltpu.VMEM((_EPT,), jnp.int32),
            pltpu.VMEM((_EPT,), jnp.float32),
            pltpu.VMEM((_N,), jnp.float32),
        ],
    )(g32, ew)

    yt, dinv = pl.pallas_call(
        _dense1_body,
        out_shape=(
            jax.ShapeDtypeStruct((_FCAT, _N), jnp.float32),
            jax.ShapeDtypeStruct((1, _N), jnp.float32),
        ),
    )(deg_parts, W_conv_z, W_conv_r, W_conv_h, node_feat)

    convT_flat = pl.kernel(
        _accum_body,
        out_type=jax.ShapeDtypeStruct((_FCAT * _N,), jnp.float32),
        mesh=mesh,
        compiler_params=sc_params,
        scratch_types=[
            pltpu.VMEM((_FPT * _N,), jnp.float32),
            pltpu.VMEM((_FPT * _N,), jnp.float32),
            pltpu.VMEM((_CHUNK,), jnp.int32),
            pltpu.VMEM((_CHUNK,), jnp.int32),
            pltpu.VMEM((_CHUNK,), jnp.float32),
            pltpu.VMEM((_CHUNK,), jnp.int32),
            pltpu.VMEM((_CHUNK,), jnp.int32),
            pltpu.VMEM((_CHUNK,), jnp.float32),
            pltpu.SemaphoreType.DMA((2,)),
        ],
    )(g32, ew, yt.reshape(-1))

    hnT, yT = pl.pallas_call(
        _dense2_body,
        out_shape=(
            jax.ShapeDtypeStruct((32, _N), jnp.float32),
            jax.ShapeDtypeStruct((1, _N), jnp.float32),
        ),
    )(
        convT_flat.reshape(_FCAT, _N), dinv, hidden_state.T,
        W_lin_z[:32].T, W_lin_z[32:].T,
        W_lin_r[:32].T, W_lin_r[32:].T,
        W_lin_h[:32].T, W_lin_h[32:].T,
        b_conv_z.reshape(32, 1), b_conv_r.reshape(32, 1), b_conv_h.reshape(32, 1),
        b_lin_z.reshape(32, 1), b_lin_r.reshape(32, 1), b_lin_h.reshape(32, 1),
        W_out.T, b_out.reshape(1, 1),
    )

    return (yT.T, hnT.T)


# speedup vs baseline: 1.0940x; 1.0901x over previous
import jax
import jax.numpy as jnp
from jax import lax
from jax.experimental import pallas as pl
from jax.experimental.pallas import tpu as pltpu
from jax.experimental.pallas import tpu_sc as plsc

_N = 10000
_E = 320000
_FCAT = 96
_NTILES = 32
_EPT = _E // _NTILES
_FPT = _FCAT // _NTILES
_CHUNK = 10000
_L = 16


def _tile_id():
    return lax.axis_index("s") * 2 + lax.axis_index("c")


def _zero_vmem(ref, nwords):
    z = jnp.zeros((_L,), ref.dtype)

    @plsc.parallel_loop(0, nwords // _L, unroll=8)
    def _(i):
        ref[pl.ds(i * _L, _L)] = z


def _deg_body(g_hbm, ew_hbm, out_hbm, col_v, ew_v, deg_v):
    wid = _tile_id()
    base = wid * _EPT
    pltpu.sync_copy(g_hbm.at[pl.ds(_E + base, _EPT)], col_v)
    pltpu.sync_copy(ew_hbm.at[pl.ds(base, _EPT)], ew_v)
    _zero_vmem(deg_v, _N)

    @plsc.parallel_loop(0, _EPT // _L, unroll=8)
    def _(i):
        c16 = col_v[pl.ds(i * _L, _L)]
        w16 = ew_v[pl.ds(i * _L, _L)]
        plsc.addupdate_scatter(deg_v, [c16], w16)
    pltpu.sync_copy(deg_v, out_hbm.at[wid])


def _accum_body(g_hbm, ew_hbm, yzr_hbm, yh_hbm, out_hbm,
                y_v, acc_v, row_v0, col_v0, ew_v0, row_v1, col_v1, ew_v1, sem):
    wid = _tile_id()
    fbase = wid * _N
    nch = _E // _CHUNK
    bufs = ((row_v0, col_v0, ew_v0), (row_v1, col_v1, ew_v1))

    def issue(ci, b):
        cbase = ci * _CHUNK
        rv, cv, wv = bufs[b]
        pltpu.async_copy(g_hbm.at[pl.ds(cbase, _CHUNK)], rv, sem.at[b])
        pltpu.async_copy(g_hbm.at[pl.ds(_E + cbase, _CHUNK)], cv, sem.at[b])
        pltpu.async_copy(ew_hbm.at[pl.ds(cbase, _CHUNK)], wv, sem.at[b])

    def drain(ci, b):
        cbase = ci * _CHUNK
        rv, cv, wv = bufs[b]
        pltpu.make_async_copy(g_hbm.at[pl.ds(cbase, _CHUNK)], rv,
                              sem.at[b]).wait()
        pltpu.make_async_copy(g_hbm.at[pl.ds(_E + cbase, _CHUNK)], cv,
                              sem.at[b]).wait()
        pltpu.make_async_copy(ew_hbm.at[pl.ds(cbase, _CHUNK)], wv,
                              sem.at[b]).wait()

    issue(0, 0)
    pltpu.sync_copy(yzr_hbm.at[pl.ds(fbase, _N)], y_v.at[pl.ds(0, _N)])
    pltpu.sync_copy(yh_hbm.at[pl.ds(fbase, _N)], y_v.at[pl.ds(_N, _N)])
    _zero_vmem(acc_v, 3 * _N)

    def outer(cc, carry):
        for b in range(2):
            ci = cc + b
            drain(ci, b)

            @pl.when(ci + 1 < nch)
            def _():
                issue(ci + 1, (b + 1) % 2)

            rv, cv, wv = bufs[b]

            @plsc.parallel_loop(0, _CHUNK // _L, unroll=8)
            def _(i):
                r16 = rv[pl.ds(i * _L, _L)]
                c16 = cv[pl.ds(i * _L, _L)]
                w16 = wv[pl.ds(i * _L, _L)]
                pzr = plsc.load_gather(y_v, [r16])
                vz, vr = plsc.unpack(
                    plsc.bitcast(pzr, jnp.bfloat16),
                    format=plsc.PackFormat.INTERLEAVED,
                    preferred_element_type=jnp.float32)
                vh = plsc.load_gather(y_v, [r16 + _N])
                plsc.addupdate_scatter(acc_v, [c16], vz * w16)
                plsc.addupdate_scatter(acc_v, [c16 + _N], vr * w16)
                plsc.addupdate_scatter(acc_v, [c16 + 2 * _N], vh * w16)

        return carry

    lax.fori_loop(0, nch // 2, lambda j, c: outer(j * 2, c), None)
    for f in range(3):
        pltpu.sync_copy(acc_v.at[pl.ds(f * _N, _N)],
                        out_hbm.at[pl.ds((f * _NTILES + wid) * _N, _N)])


def _dense1_body(degp_ref, wz_ref, wr_ref, wh_ref, x_ref,
                 yzr_ref, yh_ref, dinv_ref):
    deg = jnp.sum(degp_ref[...], axis=0, keepdims=True)
    pos = deg > 0.0
    dinv = jnp.where(pos, lax.rsqrt(jnp.where(pos, deg, 1.0)), 0.0)
    dinv_ref[...] = dinv
    x = x_ref[...]

    def xwt(w_ref):
        out = lax.dot_general(w_ref[...], x, (((0,), (1,)), ((), ())),
                              preferred_element_type=jnp.float32)
        return out * dinv

    def halfbits(v):
        return lax.bitcast_convert_type(
            v.astype(jnp.bfloat16), jnp.uint16).astype(jnp.uint32)

    packed = halfbits(xwt(wz_ref)) | (halfbits(xwt(wr_ref)) << 16)
    yzr_ref[...] = lax.bitcast_convert_type(packed, jnp.float32)
    yh_ref[...] = xwt(wh_ref)


def _dense2_body(convT_ref, dinv_ref, hT_ref,
                 azT, bzT, arT, brT, ahT, bhT,
                 bcz, bcr, bch, blz, blr, blh,
                 woT, bout, hnT_ref, yT_ref):
    dinv = dinv_ref[...]
    hT = hT_ref[...]
    czT = convT_ref[0:32, :] * dinv + bcz[...]
    crT = convT_ref[32:64, :] * dinv + bcr[...]
    chT = convT_ref[64:96, :] * dinv + bch[...]
    zT = jax.nn.sigmoid(azT[...] @ czT + bzT[...] @ hT + blz[...])
    rT = jax.nn.sigmoid(arT[...] @ crT + brT[...] @ hT + blr[...])
    htT = jnp.tanh(ahT[...] @ chT + bhT[...] @ (hT * rT) + blh[...])
    hnT = zT * hT + (1.0 - zT) * htT
    hnT_ref[...] = hnT
    yT_ref[...] = woT[...] @ jnp.maximum(hnT, 0.0) + bout[...]


def kernel(g, node_feat, edge_weight, hidden_state,
           W_conv_z, b_conv_z, W_lin_z, b_lin_z,
           W_conv_r, b_conv_r, W_lin_r, b_lin_r,
           W_conv_h, b_conv_h, W_lin_h, b_lin_h,
           W_out, b_out):
    g32 = g.astype(jnp.int32).reshape(-1)
    ew = edge_weight.astype(jnp.float32)

    mesh = plsc.VectorSubcoreMesh(core_axis_name="c", subcore_axis_name="s")
    sc_params = pltpu.CompilerParams(needs_layout_passes=False)

    deg_parts = pl.kernel(
        _deg_body,
        out_type=jax.ShapeDtypeStruct((_NTILES, _N), jnp.float32),
        mesh=mesh,
        compiler_params=sc_params,
        scratch_types=[
            pltpu.VMEM((_EPT,), jnp.int32),
            pltpu.VMEM((_EPT,), jnp.float32),
            pltpu.VMEM((_N,), jnp.float32),
        ],
    )(g32, ew)

    yzr, yh, dinv = pl.pallas_call(
        _dense1_body,
        out_shape=(
            jax.ShapeDtypeStruct((_NTILES, _N), jnp.float32),
            jax.ShapeDtypeStruct((_NTILES, _N), jnp.float32),
            jax.ShapeDtypeStruct((1, _N), jnp.float32),
        ),
    )(deg_parts, W_conv_z, W_conv_r, W_conv_h, node_feat)

    convT_flat = pl.kernel(
        _accum_body,
        out_type=jax.ShapeDtypeStruct((_FCAT * _N,), jnp.float32),
        mesh=mesh,
        compiler_params=sc_params,
        scratch_types=[
            pltpu.VMEM((2 * _N,), jnp.float32),
            pltpu.VMEM((3 * _N,), jnp.float32),
            pltpu.VMEM((_CHUNK,), jnp.int32),
            pltpu.VMEM((_CHUNK,), jnp.int32),
            pltpu.VMEM((_CHUNK,), jnp.float32),
            pltpu.VMEM((_CHUNK,), jnp.int32),
            pltpu.VMEM((_CHUNK,), jnp.int32),
            pltpu.VMEM((_CHUNK,), jnp.float32),
            pltpu.SemaphoreType.DMA((2,)),
        ],
    )(g32, ew, yzr.reshape(-1), yh.reshape(-1))

    hnT, yT = pl.pallas_call(
        _dense2_body,
        out_shape=(
            jax.ShapeDtypeStruct((32, _N), jnp.float32),
            jax.ShapeDtypeStruct((1, _N), jnp.float32),
        ),
    )(
        convT_flat.reshape(_FCAT, _N), dinv, hidden_state.T,
        W_lin_z[:32].T, W_lin_z[32:].T,
        W_lin_r[:32].T, W_lin_r[32:].T,
        W_lin_h[:32].T, W_lin_h[32:].T,
        b_conv_z.reshape(32, 1), b_conv_r.reshape(32, 1), b_conv_h.reshape(32, 1),
        b_lin_z.reshape(32, 1), b_lin_r.reshape(32, 1), b_lin_h.reshape(32, 1),
        W_out.T, b_out.reshape(1, 1),
    )

    return (yT.T, hnT.T)
